# baseline (speedup 1.0000x reference)
import functools

import jax
import jax.numpy as jnp
from jax import lax
from jax.experimental import pallas as pl
from jax.experimental.pallas import tpu as pltpu
from jax.experimental.pallas import tpu_sc as plsc

_H = 512
_W = 512
_B = 16
_P = 1024
_K = 15
_ROWS = 128
_NRB = _H // _ROWS
_NCHUNK = _B * _NRB
_SENT = 1 << 20
_DUMP = _ROWS


def _make_sc_call():
    info = plsc.get_sparse_core_info()
    nc, ns = info.num_cores, info.num_subcores
    nw = nc * ns
    npass = _NCHUNK // nw
    mesh = plsc.VectorSubcoreMesh(core_axis_name="c", subcore_axis_name="s")

    @functools.partial(
        pl.kernel,
        mesh=mesh,
        compiler_params=pltpu.CompilerParams(needs_layout_passes=False),
        out_type=jax.ShapeDtypeStruct((_B, 1, _H, _W), jnp.float32),
        scratch_types=[
            pltpu.VMEM((_P,), jnp.int32),
            pltpu.VMEM((_K * _K + 15, 16), jnp.float32),
            pltpu.VMEM((_ROWS + 1, _W), jnp.float32),
            pltpu.VMEM((_P + 16,), jnp.int32),
        ],
    )
    def stamp(pk_hbm, wsp_hbm, out_hbm, pkv, wsp, acc, plo):
        wid = lax.axis_index("s") * nc + lax.axis_index("c")
        pltpu.sync_copy(wsp_hbm, wsp)

        zv = wsp[_K * _K]
        sentv = jnp.full((16,), _SENT, jnp.int32)
        dumpv = jnp.full((16,), _DUMP, jnp.int32)
        zerov = wsp[_K * _K]

        def pass_body(ps, _unused):
            chunk = wid + ps * nw
            b = chunk // _NRB
            rb = chunk % _NRB
            r0 = rb * _ROWS

            pltpu.sync_copy(pk_hbm.at[b], pkv)

            def zbody(r, _):
                for j in range(_W // 16):
                    acc[r, pl.ds(j * 16, 16)] = zv
                return _

            lax.fori_loop(0, _ROWS, zbody, None)

            def cbody(g, off):
                pvec = pkv[pl.ds(g * 16, 16)]
                cxv = lax.shift_right_logical(pvec, 10)
                rbv = cxv - (7 + r0)
                ov = (rbv + (_K - 1)).astype(jnp.uint32) <= _ROWS + _K - 2
                plsc.store_compressed(plo.at[pl.ds(off, 16)], pvec, mask=ov)
                return off + plsc.all_reduce_population_count(ov)[0]

            off = lax.fori_loop(0, _P // 16, cbody, jnp.int32(0))
            plo[pl.ds(off, 16)] = sentv

            def sbody(g, _):
                @pl.when(g * 16 <= off)
                def _grp():
                    pvec = plo[pl.ds(g * 16, 16)]
                    cxv = lax.shift_right_logical(pvec, 10)
                    cyv = pvec & 1023
                    rowv = cxv - (7 + r0)
                    colv = cyv - 7
                    masks = []
                    ct = colv
                    for t in range(_K):
                        masks.append(ct.astype(jnp.uint32) < _W)
                        if t < _K - 1:
                            ct = ct + 1
                    for k in range(_K):
                        vals_k = [wsp[k * _K + t] for t in range(_K)]
                        rk = rowv + k
                        srow = jnp.where(rk.astype(jnp.uint32) < _ROWS, rk, dumpv)
                        ctv = colv
                        for t in range(_K):
                            sval = jnp.where(masks[t], vals_k[t], zerov)
                            sct = jnp.where(masks[t], ctv, 0)
                            plsc.addupdate_scatter(acc, [srow, sct], sval)
                            if t < _K - 1:
                                ctv = ctv + 1
                return _

            lax.fori_loop(0, _P // 16 + 1, sbody, None)

            pltpu.sync_copy(
                acc.at[pl.ds(0, _ROWS)], out_hbm.at[b, 0, pl.ds(r0, _ROWS)]
            )
            return _unused

        lax.fori_loop(0, npass, pass_body, None)

    return stamp


def kernel(batch_images, batch_labels, sigma):
    del batch_images
    ax = jnp.arange(_K, dtype=jnp.float32) - (_K // 2)
    g = jnp.exp(-(ax * ax) / (2.0 * sigma * sigma))
    g = g / jnp.sum(g)
    taps = (g[:, None] * g[None, :]).reshape(_K * _K)
    wsp = jnp.zeros((_K * _K + 15, 16), jnp.float32)
    wsp = wsp.at[: _K * _K].set(jnp.broadcast_to(taps[:, None], (_K * _K, 16)))

    c = jnp.trunc(batch_labels.astype(jnp.float32) - (_K / 2)).astype(jnp.int32) + (_K // 2)
    packed = c[:, :, 0] * 1024 + c[:, :, 1]

    return _make_sc_call()(packed, wsp)


# speedup vs baseline: 1.0475x; 1.0475x over previous
import functools

import jax
import jax.numpy as jnp
from jax import lax
from jax.experimental import pallas as pl
from jax.experimental.pallas import tpu as pltpu
from jax.experimental.pallas import tpu_sc as plsc

_H = 512
_W = 512
_B = 16
_P = 1024
_K = 15
_ROWS = 128
_NRB = _H // _ROWS
_NCHUNK = _B * _NRB
_SENT = 1 << 20
_DUMP = _ROWS


def _make_sc_call():
    info = plsc.get_sparse_core_info()
    nc, ns = info.num_cores, info.num_subcores
    nw = nc * ns
    npass = _NCHUNK // nw
    mesh = plsc.VectorSubcoreMesh(core_axis_name="c", subcore_axis_name="s")

    @functools.partial(
        pl.kernel,
        mesh=mesh,
        compiler_params=pltpu.CompilerParams(needs_layout_passes=False),
        out_type=jax.ShapeDtypeStruct((_B, 1, _H, _W), jnp.float32),
        scratch_types=[
            pltpu.VMEM((_P,), jnp.int32),
            pltpu.VMEM((_K * _K + 15, 16), jnp.float32),
            pltpu.VMEM((_ROWS + 1, _W), jnp.float32),
            pltpu.VMEM((_P + 16,), jnp.int32),
        ],
    )
    def stamp(pk_hbm, wsp_hbm, out_hbm, pkv, wsp, acc, plo):
        wid = lax.axis_index("s") * nc + lax.axis_index("c")
        pltpu.sync_copy(wsp_hbm, wsp)

        zv = wsp[_K * _K]
        sentv = jnp.full((16,), _SENT, jnp.int32)
        dumpv = jnp.full((16,), _DUMP, jnp.int32)

        def pass_body(ps, _unused):
            chunk = wid + ps * nw
            b = chunk // _NRB
            rb = chunk % _NRB
            r0 = rb * _ROWS

            pltpu.sync_copy(pk_hbm.at[b], pkv)

            def zbody(r, _):
                for j in range(_W // 16):
                    acc[r, pl.ds(j * 16, 16)] = zv
                return _

            lax.fori_loop(0, _ROWS, zbody, None)

            def cbody(g, off):
                pvec = pkv[pl.ds(g * 16, 16)]
                cxv = lax.shift_right_logical(pvec, 10)
                rbv = cxv - (7 + r0)
                ov = (rbv + (_K - 1)).astype(jnp.uint32) <= _ROWS + _K - 2
                plsc.store_compressed(plo.at[pl.ds(off, 16)], pvec, mask=ov)
                return off + plsc.all_reduce_population_count(ov)[0]

            off = lax.fori_loop(0, _P // 16, cbody, jnp.int32(0))
            plo[pl.ds(off, 16)] = sentv

            def sbody(g, _):
                @pl.when(g * 16 <= off)
                def _grp():
                    pvec = plo[pl.ds(g * 16, 16)]
                    cxv = lax.shift_right_logical(pvec, 10)
                    cyv = pvec & 1023
                    rowv = cxv - (7 + r0)
                    colv = cyv - 7
                    masks = []
                    ct = colv
                    for t in range(_K):
                        masks.append(ct.astype(jnp.uint32) < _W)
                        if t < _K - 1:
                            ct = ct + 1
                    for k in range(_K):
                        vals_k = [wsp[k * _K + t] for t in range(_K)]
                        rk = rowv + k
                        srow = jnp.where(rk.astype(jnp.uint32) < _ROWS, rk, dumpv)
                        ctv = colv
                        for t in range(_K):
                            plsc.addupdate_scatter(
                                acc, [srow, ctv], vals_k[t], mask=masks[t]
                            )
                            if t < _K - 1:
                                ctv = ctv + 1
                return _

            lax.fori_loop(0, _P // 16 + 1, sbody, None)

            pltpu.sync_copy(
                acc.at[pl.ds(0, _ROWS)], out_hbm.at[b, 0, pl.ds(r0, _ROWS)]
            )
            return _unused

        lax.fori_loop(0, npass, pass_body, None)

    return stamp


def kernel(batch_images, batch_labels, sigma):
    del batch_images
    ax = jnp.arange(_K, dtype=jnp.float32) - (_K // 2)
    g = jnp.exp(-(ax * ax) / (2.0 * sigma * sigma))
    g = g / jnp.sum(g)
    taps = (g[:, None] * g[None, :]).reshape(_K * _K)
    wsp = jnp.zeros((_K * _K + 15, 16), jnp.float32)
    wsp = wsp.at[: _K * _K].set(jnp.broadcast_to(taps[:, None], (_K * _K, 16)))

    c = jnp.trunc(batch_labels.astype(jnp.float32) - (_K / 2)).astype(jnp.int32) + (_K // 2)
    packed = c[:, :, 0] * 1024 + c[:, :, 1]

    return _make_sc_call()(packed, wsp)
